# baseline (speedup 1.0000x reference)
import functools

import jax
import jax.numpy as jnp
from jax import lax
from jax.experimental import pallas as pl
from jax.experimental.pallas import tpu as pltpu
from jax.experimental.pallas import tpu_sc as plsc

E = 64
D = 768
F = 768
K = 512
T = 32768
NW = 32
EPW = E // NW
NV = T // 16


def _router_body(x_ref, w_ref, logits_ref, probt_ref):
    x = x_ref[...]
    w = w_ref[...]
    logits = jax.lax.dot_general(x, w, (((1,), (1,)), ((), ())),
                                 preferred_element_type=jnp.float32)
    logits_ref[...] = logits
    m = jnp.max(logits, axis=-1, keepdims=True)
    p = jnp.exp(logits - m)
    p = p / jnp.sum(p, axis=-1, keepdims=True)
    probt_ref[...] = p.T


def _router(x, router_w):
    bT = 2048
    return pl.pallas_call(
        _router_body,
        grid=(T // bT,),
        in_specs=[
            pl.BlockSpec((bT, D), lambda i: (i, 0)),
            pl.BlockSpec((E, D), lambda i: (0, 0)),
        ],
        out_specs=[
            pl.BlockSpec((bT, E), lambda i: (i, 0)),
            pl.BlockSpec((E, bT), lambda i: (0, i)),
        ],
        out_shape=[
            jax.ShapeDtypeStruct((T, E), jnp.float32),
            jax.ShapeDtypeStruct((E, T), jnp.float32),
        ],
    )(x, router_w)


def _mlp_body(x_ref, w1_ref, b1_ref, w2_ref, b2_ref, wt_ref, out_ref):
    x = x_ref[0]
    h = jax.lax.dot_general(x, w1_ref[0], (((1,), (0,)), ((), ())),
                            preferred_element_type=jnp.float32)
    h = jax.nn.gelu(h + b1_ref[0])
    o = jax.lax.dot_general(h, w2_ref[0], (((1,), (0,)), ((), ())),
                            preferred_element_type=jnp.float32)
    o = o + b2_ref[0]
    out_ref[...] = wt_ref[0].T * o


def _mlp(x_in, W1, b1, W2, b2, wts):
    return pl.pallas_call(
        _mlp_body,
        grid=(E,),
        in_specs=[
            pl.BlockSpec((1, K, D), lambda e: (e, 0, 0)),
            pl.BlockSpec((1, D, F), lambda e: (e, 0, 0)),
            pl.BlockSpec((1, 1, F), lambda e: (e, 0, 0)),
            pl.BlockSpec((1, F, D), lambda e: (e, 0, 0)),
            pl.BlockSpec((1, 1, D), lambda e: (e, 0, 0)),
            pl.BlockSpec((1, 1, K), lambda e: (e, 0, 0)),
        ],
        out_specs=pl.BlockSpec((K, D), lambda e: (e, 0)),
        out_shape=jax.ShapeDtypeStruct((E * K + 32, D), jnp.float32),
    )(x_in, W1, b1.reshape(E, 1, F), W2, b2.reshape(E, 1, D),
      wts.reshape(E, 1, K))


def _scan_desc(hist_ref, nvregs, base, kk):
    lanes = lax.iota(jnp.int32, 16)

    def cond(st):
        j, carry, b = st
        return jnp.logical_and(b < 0, j >= 0)

    def body(st):
        j, carry, b = st
        v = hist_ref[pl.ds(j * 16, 16)]
        rv = lax.rev(v, dimensions=(0,))
        cd = plsc.cumsum(rv) + carry
        m = (cd + base) >= kk
        found = jnp.sum(m.astype(jnp.int32)) > 0
        i0 = jnp.min(jnp.where(m, lanes, jnp.int32(16)))
        bnew = 16 * j + 15 - i0
        return (j - 1, carry + jnp.sum(v),
                jnp.where(found, bnew, jnp.int32(-1)))

    st = (jnp.int32(nvregs - 1), jnp.int32(0), jnp.int32(-1))
    _, _, b = lax.while_loop(cond, body, st)
    return jnp.maximum(b, 0)


def _masked_sum_above(hist_ref, nvregs, b):
    lanes = lax.iota(jnp.int32, 16)

    def body(j, acc):
        v = hist_ref[pl.ds(j * 16, 16)]
        m = (lanes + 16 * j) > b
        return acc + jnp.sum(jnp.where(m, v, 0))

    return lax.fori_loop(0, nvregs, body, jnp.int32(0))


def _topk_body(probt_hbm, idx_hbm, wts_hbm, be_hbm, pbuf, h1, h8a, h8b,
               seli, selw, bbuf):
    wid = lax.axis_index("s") * 2 + lax.axis_index("c")
    lanes = lax.iota(jnp.int32, 16)
    zeros16 = jnp.zeros((16,), jnp.int32)
    ones16 = jnp.ones((16,), jnp.int32)
    kk = jnp.int32(K)

    def per_expert(eo, _):
        e = wid * EPW + eo
        pltpu.sync_copy(probt_hbm.at[e], pbuf)

        def zero1(j, _):
            h1[pl.ds(j * 16, 16)] = zeros16
            return 0
        lax.fori_loop(0, 1024, zero1, 0)

        def hist1(j, _):
            bits = pbuf[pl.ds(j * 16, 16)]
            plsc.addupdate_scatter(h1, [bits >> 16], ones16)
            return 0
        lax.fori_loop(0, NV, hist1, 0)
        b1 = _scan_desc(h1, 1024, jnp.int32(0), kk)
        cgt1 = _masked_sum_above(h1, 1024, b1)

        def zero2(j, _):
            h8a[pl.ds(j * 16, 16)] = zeros16
            h8b[pl.ds(j * 16, 16)] = zeros16
            return 0
        lax.fori_loop(0, 16, zero2, 0)

        def hist2(j, _):
            bits = pbuf[pl.ds(j * 16, 16)]
            m = (bits >> 16) == b1
            plsc.addupdate_scatter(h8a, [(bits >> 8) & 0xFF], ones16, mask=m)
            return 0
        lax.fori_loop(0, NV, hist2, 0)
        b2 = _scan_desc(h8a, 16, cgt1, kk)
        cgt2 = cgt1 + _masked_sum_above(h8a, 16, b2)

        pfx = (b1 << 8) | b2

        def hist3(j, _):
            bits = pbuf[pl.ds(j * 16, 16)]
            m = (bits >> 8) == pfx
            plsc.addupdate_scatter(h8b, [bits & 0xFF], ones16, mask=m)
            return 0
        lax.fori_loop(0, NV, hist3, 0)
        b3 = _scan_desc(h8b, 16, cgt2, kk)
        n_gt = cgt2 + _masked_sum_above(h8b, 16, b3)
        vstar = (pfx << 8) | b3

        def compact(j, st):
            cnt, neq = st
            bits = pbuf[pl.ds(j * 16, 16)]
            mgt = bits > vstar
            meq = bits == vstar
            eqrank = plsc.cumsum(meq.astype(jnp.int32)) - meq.astype(jnp.int32)
            allow = jnp.logical_and(meq, eqrank < neq)
            msel = jnp.logical_or(mgt, allow)
            tok = lanes + 16 * j
            plsc.store_compressed(seli.at[pl.ds(cnt, 16)], tok, mask=msel)
            plsc.store_compressed(selw.at[pl.ds(cnt, 16)], bits, mask=msel)
            return (cnt + jnp.sum(msel.astype(jnp.int32)),
                    neq - jnp.sum(allow.astype(jnp.int32)))
        lax.fori_loop(0, NV, compact, (jnp.int32(0), kk - n_gt))

        pltpu.sync_copy(seli.at[pl.ds(0, K)], idx_hbm.at[e])
        pltpu.sync_copy(selw.at[pl.ds(0, K)], wts_hbm.at[e])

        def zero4(j, _):
            h1[pl.ds(j * 16, 16)] = zeros16
            return 0
        lax.fori_loop(0, 34, zero4, 0)

        def histw(j, _):
            v = seli[pl.ds(j * 16, 16)]
            plsc.addupdate_scatter(h1, [v >> 6], ones16)
            return 0
        lax.fori_loop(0, K // 16, histw, 0)

        def prefw(j, carry):
            v = h1[pl.ds(j * 16, 16)]
            incl = plsc.cumsum(v)
            bbuf[pl.ds(j * 16, 16)] = incl - v + carry
            return carry + jnp.sum(v)
        lax.fori_loop(0, 33, prefw, jnp.int32(0))
        pltpu.sync_copy(bbuf.at[pl.ds(0, 528)], be_hbm.at[e])
        return 0

    lax.fori_loop(0, EPW, per_expert, 0)


def _topk_sc(probt):
    mesh = plsc.VectorSubcoreMesh(core_axis_name="c", subcore_axis_name="s",
                                  num_cores=2, num_subcores=16)
    f = pl.kernel(
        _topk_body,
        out_type=[
            jax.ShapeDtypeStruct((E, K), jnp.int32),
            jax.ShapeDtypeStruct((E, K), jnp.int32),
            jax.ShapeDtypeStruct((E, 528), jnp.int32),
        ],
        mesh=mesh,
        compiler_params=pltpu.CompilerParams(needs_layout_passes=False),
        scratch_types=[
            pltpu.VMEM((T,), jnp.int32),
            pltpu.VMEM((16384,), jnp.int32),
            pltpu.VMEM((256,), jnp.int32),
            pltpu.VMEM((256,), jnp.int32),
            pltpu.VMEM((K + 16,), jnp.int32),
            pltpu.VMEM((K + 16,), jnp.int32),
            pltpu.VMEM((528,), jnp.int32),
        ],
    )
    sel, wbits, be = f(lax.bitcast_convert_type(probt, jnp.int32))
    return sel, lax.bitcast_convert_type(wbits, jnp.float32), be


WTOK = 64
NWIN = T // WTOK
WPS = NWIN // 32


def _transpose_body(in_ref, out_ref):
    out_ref[...] = in_ref[...].T


def _transpose_bounds(be):
    return pl.pallas_call(
        _transpose_body,
        out_shape=jax.ShapeDtypeStruct((528, E), jnp.int32),
    )(be)


def _combine_body(contrib_hbm, sel_hbm, bt_hbm, out_hbm, acc, sbuf, btb, lidx):
    wid = lax.axis_index("s") * 2 + lax.axis_index("c")
    lanes = lax.iota(jnp.int32, 16)
    zeros16f = jnp.zeros((16,), jnp.float32)
    pltpu.sync_copy(sel_hbm, sbuf)

    def window(p, _):
        w = wid * WPS + p
        tb = w * WTOK

        def zr(j, _):
            acc[j // (D // 16), pl.ds((j % (D // 16)) * 16, 16)] = zeros16f
            return 0
        lax.fori_loop(0, 65 * (D // 16), zr, 0)

        pltpu.sync_copy(bt_hbm.at[pl.ds(w, 2)], btb)

        def per_expert(e, _):
            qq = (e >> 4) * 16
            ll = e & 15
            lov = btb[0, pl.ds(qq, 16)]
            hiv = btb[1, pl.ds(qq, 16)]
            lo = jnp.sum(jnp.where(lanes == ll, lov, 0))
            hi = jnp.sum(jnp.where(lanes == ll, hiv, 0))
            lo_al = (lo >> 4) << 4
            nsub = jnp.where(hi > lo, (hi - lo_al + 15) >> 4, 0)

            def sub(t, _):
                off = lo_al + t * 16
                tv = sbuf[pl.ds(e * K + off, 16)]
                pos = off + lanes
                valid = jnp.logical_and(pos >= lo, pos < hi)
                lidx[0, pl.ds(0, 16)] = jnp.where(valid, tv - tb,
                                                  jnp.int32(WTOK))
                pltpu.sync_copy(contrib_hbm.at[pl.ds(e * K + off, 16)],
                                acc.at[lidx.at[0]], add=True)
                return 0
            lax.fori_loop(0, nsub, sub, 0)
            return 0
        lax.fori_loop(0, E, per_expert, 0)

        pltpu.sync_copy(acc.at[pl.ds(0, WTOK)],
                        out_hbm.at[pl.ds(tb, WTOK)])
        return 0

    lax.fori_loop(0, WPS, window, 0)


def _combine_sc(contrib, selflat, bt):
    mesh = plsc.VectorSubcoreMesh(core_axis_name="c", subcore_axis_name="s",
                                  num_cores=2, num_subcores=16)
    f = pl.kernel(
        _combine_body,
        out_type=jax.ShapeDtypeStruct((T, D), jnp.float32),
        mesh=mesh,
        compiler_params=pltpu.CompilerParams(needs_layout_passes=False),
        scratch_types=[
            pltpu.VMEM((80, D), jnp.float32),
            pltpu.VMEM((E * K,), jnp.int32),
            pltpu.VMEM((2, E), jnp.int32),
            pltpu.VMEM((1, 16), jnp.int32),
        ],
    )
    return f(contrib, selflat, bt)


GCHK = 128
GCPW = (E * K) // (32 * GCHK)


def _gather_body(x_hbm, sel4_hbm, xin_hbm, idxrow, rowsbuf, sem):
    wid = lax.axis_index("s") * 2 + lax.axis_index("c")

    def chunk(c, _):
        r = wid * GCPW + c
        pltpu.sync_copy(sel4_hbm.at[r], idxrow)
        pltpu.async_copy(x_hbm.at[idxrow], rowsbuf, sem).wait()
        pltpu.sync_copy(rowsbuf, xin_hbm.at[pl.ds(r * GCHK, GCHK)])
        return 0
    lax.fori_loop(0, GCPW, chunk, 0)


def _gather_sc(x, sel):
    mesh = plsc.VectorSubcoreMesh(core_axis_name="c", subcore_axis_name="s",
                                  num_cores=2, num_subcores=16)
    f = pl.kernel(
        _gather_body,
        out_type=jax.ShapeDtypeStruct((E * K, D), jnp.float32),
        mesh=mesh,
        compiler_params=pltpu.CompilerParams(needs_layout_passes=False),
        scratch_types=[
            pltpu.VMEM((GCHK,), jnp.int32),
            pltpu.VMEM((GCHK, D), jnp.float32),
            pltpu.SemaphoreType.DMA,
        ],
    )
    return f(x, sel.reshape(-1, GCHK))


def kernel(inputs, router_w, W1, b1, W2, b2):
    B, S, _ = inputs.shape
    x = inputs.reshape(-1, D)
    logits, probt = _router(x, router_w)
    sel, wts, be = _topk_sc(probt)
    x_in = _gather_sc(x, sel).reshape(E, K, D)
    contrib = _mlp(x_in, W1, b1, W2, b2, wts)
    del be
    out = jnp.zeros((T, D), jnp.float32).at[sel.reshape(-1)].add(
        contrib[:E * K])
    return out.reshape(B, S, D), logits


# speedup vs baseline: 1.0129x; 1.0129x over previous
import functools

import jax
import jax.numpy as jnp
from jax import lax
from jax.experimental import pallas as pl
from jax.experimental.pallas import tpu as pltpu
from jax.experimental.pallas import tpu_sc as plsc

E = 64
D = 768
F = 768
K = 512
T = 32768
NW = 32
EPW = E // NW
NV = T // 16


def _router_body(x_ref, w_ref, logits_ref, probt_ref):
    x = x_ref[...]
    w = w_ref[...]
    logits = jax.lax.dot_general(x, w, (((1,), (1,)), ((), ())),
                                 preferred_element_type=jnp.float32)
    logits_ref[...] = logits
    m = jnp.max(logits, axis=-1, keepdims=True)
    p = jnp.exp(logits - m)
    p = p / jnp.sum(p, axis=-1, keepdims=True)
    probt_ref[...] = p.T


def _router(x, router_w):
    bT = 2048
    return pl.pallas_call(
        _router_body,
        grid=(T // bT,),
        in_specs=[
            pl.BlockSpec((bT, D), lambda i: (i, 0)),
            pl.BlockSpec((E, D), lambda i: (0, 0)),
        ],
        out_specs=[
            pl.BlockSpec((bT, E), lambda i: (i, 0)),
            pl.BlockSpec((E, bT), lambda i: (0, i)),
        ],
        out_shape=[
            jax.ShapeDtypeStruct((T, E), jnp.float32),
            jax.ShapeDtypeStruct((E, T), jnp.float32),
        ],
    )(x, router_w)


def _mlp_body(x_ref, w1_ref, b1_ref, w2_ref, b2_ref, wt_ref, out_ref):
    x = x_ref[0].astype(jnp.bfloat16)
    h = jax.lax.dot_general(x, w1_ref[0].astype(jnp.bfloat16),
                            (((1,), (0,)), ((), ())),
                            preferred_element_type=jnp.float32)
    h = jax.nn.gelu(h + b1_ref[0])
    o = jax.lax.dot_general(h.astype(jnp.bfloat16),
                            w2_ref[0].astype(jnp.bfloat16),
                            (((1,), (0,)), ((), ())),
                            preferred_element_type=jnp.float32)
    o = o + b2_ref[0]
    out_ref[...] = wt_ref[0].T * o


def _mlp(x_in, W1, b1, W2, b2, wts):
    return pl.pallas_call(
        _mlp_body,
        grid=(E,),
        in_specs=[
            pl.BlockSpec((1, K, D), lambda e: (e, 0, 0)),
            pl.BlockSpec((1, D, F), lambda e: (e, 0, 0)),
            pl.BlockSpec((1, 1, F), lambda e: (e, 0, 0)),
            pl.BlockSpec((1, F, D), lambda e: (e, 0, 0)),
            pl.BlockSpec((1, 1, D), lambda e: (e, 0, 0)),
            pl.BlockSpec((1, 1, K), lambda e: (e, 0, 0)),
        ],
        out_specs=pl.BlockSpec((K, D), lambda e: (e, 0)),
        out_shape=jax.ShapeDtypeStruct((E * K + 32, D), jnp.float32),
    )(x_in, W1, b1.reshape(E, 1, F), W2, b2.reshape(E, 1, D),
      wts.reshape(E, 1, K))


def _scan_desc(hist_ref, nvregs, base, kk):
    lanes = lax.iota(jnp.int32, 16)

    def cond(st):
        j, carry, b = st
        return jnp.logical_and(b < 0, j >= 0)

    def body(st):
        j, carry, b = st
        v = hist_ref[pl.ds(j * 16, 16)]
        rv = lax.rev(v, dimensions=(0,))
        cd = plsc.cumsum(rv) + carry
        m = (cd + base) >= kk
        found = jnp.sum(m.astype(jnp.int32)) > 0
        i0 = jnp.min(jnp.where(m, lanes, jnp.int32(16)))
        bnew = 16 * j + 15 - i0
        return (j - 1, carry + jnp.sum(v),
                jnp.where(found, bnew, jnp.int32(-1)))

    st = (jnp.int32(nvregs - 1), jnp.int32(0), jnp.int32(-1))
    _, _, b = lax.while_loop(cond, body, st)
    return jnp.maximum(b, 0)


def _masked_sum_above(hist_ref, nvregs, b):
    lanes = lax.iota(jnp.int32, 16)

    def body(j, acc):
        v = hist_ref[pl.ds(j * 16, 16)]
        m = (lanes + 16 * j) > b
        return acc + jnp.sum(jnp.where(m, v, 0))

    return lax.fori_loop(0, nvregs, body, jnp.int32(0))


def _topk_body(probt_hbm, idx_hbm, wts_hbm, pbuf, h1, h8a, h8b, seli, selw):
    wid = lax.axis_index("s") * 2 + lax.axis_index("c")
    lanes = lax.iota(jnp.int32, 16)
    zeros16 = jnp.zeros((16,), jnp.int32)
    ones16 = jnp.ones((16,), jnp.int32)
    kk = jnp.int32(K)

    def per_expert(eo, _):
        e = wid * EPW + eo
        pltpu.sync_copy(probt_hbm.at[e], pbuf)

        def zero1(j, _):
            for u in range(4):
                h1[pl.ds(j * 64 + u * 16, 16)] = zeros16
            return 0
        lax.fori_loop(0, 256, zero1, 0)

        def hist1(j, _):
            for u in range(4):
                bits = pbuf[pl.ds(j * 64 + u * 16, 16)]
                plsc.addupdate_scatter(h1, [bits >> 16], ones16)
            return 0
        lax.fori_loop(0, NV // 4, hist1, 0)
        b1 = _scan_desc(h1, 1024, jnp.int32(0), kk)
        cgt1 = _masked_sum_above(h1, 1024, b1)

        def zero2(j, _):
            h8a[pl.ds(j * 16, 16)] = zeros16
            h8b[pl.ds(j * 16, 16)] = zeros16
            return 0
        lax.fori_loop(0, 16, zero2, 0)

        def hist2(j, _):
            for u in range(4):
                bits = pbuf[pl.ds(j * 64 + u * 16, 16)]
                m = (bits >> 16) == b1
                plsc.addupdate_scatter(h8a, [(bits >> 8) & 0xFF], ones16,
                                       mask=m)
            return 0
        lax.fori_loop(0, NV // 4, hist2, 0)
        b2 = _scan_desc(h8a, 16, cgt1, kk)
        cgt2 = cgt1 + _masked_sum_above(h8a, 16, b2)

        pfx = (b1 << 8) | b2

        def hist3(j, _):
            for u in range(4):
                bits = pbuf[pl.ds(j * 64 + u * 16, 16)]
                m = (bits >> 8) == pfx
                plsc.addupdate_scatter(h8b, [bits & 0xFF], ones16, mask=m)
            return 0
        lax.fori_loop(0, NV // 4, hist3, 0)
        b3 = _scan_desc(h8b, 16, cgt2, kk)
        n_gt = cgt2 + _masked_sum_above(h8b, 16, b3)
        vstar = (pfx << 8) | b3

        def compact(j, st):
            cnt, neq = st
            for u in range(4):
                bits = pbuf[pl.ds(j * 64 + u * 16, 16)]
                mgt = bits > vstar
                meq = bits == vstar
                eqrank = (plsc.cumsum(meq.astype(jnp.int32))
                          - meq.astype(jnp.int32))
                allow = jnp.logical_and(meq, eqrank < neq)
                msel = jnp.logical_or(mgt, allow)
                tok = lanes + 16 * (4 * j + u)
                plsc.store_compressed(seli.at[pl.ds(cnt, 16)], tok, mask=msel)
                plsc.store_compressed(selw.at[pl.ds(cnt, 16)], bits, mask=msel)
                cnt = cnt + jnp.sum(msel.astype(jnp.int32))
                neq = neq - jnp.sum(allow.astype(jnp.int32))
            return (cnt, neq)
        lax.fori_loop(0, NV // 4, compact, (jnp.int32(0), kk - n_gt))

        pltpu.sync_copy(seli.at[pl.ds(0, K)], idx_hbm.at[e])
        pltpu.sync_copy(selw.at[pl.ds(0, K)], wts_hbm.at[e])

        return 0

    lax.fori_loop(0, EPW, per_expert, 0)


def _topk_sc(probt):
    mesh = plsc.VectorSubcoreMesh(core_axis_name="c", subcore_axis_name="s",
                                  num_cores=2, num_subcores=16)
    f = pl.kernel(
        _topk_body,
        out_type=[
            jax.ShapeDtypeStruct((E, K), jnp.int32),
            jax.ShapeDtypeStruct((E, K), jnp.int32),
        ],
        mesh=mesh,
        compiler_params=pltpu.CompilerParams(needs_layout_passes=False),
        scratch_types=[
            pltpu.VMEM((T,), jnp.int32),
            pltpu.VMEM((16384,), jnp.int32),
            pltpu.VMEM((256,), jnp.int32),
            pltpu.VMEM((256,), jnp.int32),
            pltpu.VMEM((K + 16,), jnp.int32),
            pltpu.VMEM((K + 16,), jnp.int32),
        ],
    )
    sel, wbits = f(lax.bitcast_convert_type(probt, jnp.int32))
    return sel, lax.bitcast_convert_type(wbits, jnp.float32)


GCHK = 128
GCPW = (E * K) // (32 * GCHK)


def _gather_body(x_hbm, sel4_hbm, xin_hbm, idxrow, rowsbuf, sem):
    wid = lax.axis_index("s") * 2 + lax.axis_index("c")

    def chunk(c, _):
        r = wid * GCPW + c
        pltpu.sync_copy(sel4_hbm.at[r], idxrow)
        pltpu.async_copy(x_hbm.at[idxrow], rowsbuf, sem).wait()
        pltpu.sync_copy(rowsbuf, xin_hbm.at[pl.ds(r * GCHK, GCHK)])
        return 0
    lax.fori_loop(0, GCPW, chunk, 0)


def _gather_sc(x, sel):
    mesh = plsc.VectorSubcoreMesh(core_axis_name="c", subcore_axis_name="s",
                                  num_cores=2, num_subcores=16)
    f = pl.kernel(
        _gather_body,
        out_type=jax.ShapeDtypeStruct((E * K, D), jnp.float32),
        mesh=mesh,
        compiler_params=pltpu.CompilerParams(needs_layout_passes=False),
        scratch_types=[
            pltpu.VMEM((GCHK,), jnp.int32),
            pltpu.VMEM((GCHK, D), jnp.float32),
            pltpu.SemaphoreType.DMA,
        ],
    )
    return f(x, sel.reshape(-1, GCHK))


def kernel(inputs, router_w, W1, b1, W2, b2):
    B, S, _ = inputs.shape
    x = inputs.reshape(-1, D)
    logits, probt = _router(x, router_w)
    sel, wts = _topk_sc(probt)
    x_in = _gather_sc(x, sel).reshape(E, K, D)
    contrib = _mlp(x_in, W1, b1, W2, b2, wts)
    out = jnp.zeros((T, D), jnp.float32).at[sel.reshape(-1)].add(
        contrib[:E * K])
    return out.reshape(B, S, D), logits
